# Initial kernel scaffold; baseline (speedup 1.0000x reference)
import jax
import jax.numpy as jnp
from jax.experimental import pallas as pl


def kernel(event_type_ids, src_ids, src_mask, dst_ids, dst_mask, event_embeddings, event_timestamps, memory, last_update, time_w, time_b):
    raise NotImplementedError("write your pallas kernel here")


# speedup vs baseline: 3.7940x; 3.7940x over previous
import functools

import jax
import jax.numpy as jnp
from jax import lax
from jax.experimental import pallas as pl
from jax.experimental.pallas import tpu as pltpu
from jax.experimental.pallas import tpu_sc as plsc

_NUM_CORES = 2
_NUM_SUBCORES = 16
_NUM_WORKERS = _NUM_CORES * _NUM_SUBCORES
_CHUNK = 128


@functools.lru_cache(maxsize=None)
def _make_sc_gather(E, N, H):
    n_chunks = E // _CHUNK
    iters = -(-n_chunks // _NUM_WORKERS)
    mesh = plsc.VectorSubcoreMesh(core_axis_name="c", subcore_axis_name="s")
    f32 = jnp.float32

    @functools.partial(
        pl.kernel,
        mesh=mesh,
        out_type=[
            jax.ShapeDtypeStruct((E, H), f32),
            jax.ShapeDtypeStruct((E, H), f32),
            jax.ShapeDtypeStruct((E,), f32),
            jax.ShapeDtypeStruct((E,), f32),
        ],
        scratch_types=[
            pltpu.VMEM((_CHUNK,), jnp.int32),
            pltpu.VMEM((_CHUNK,), jnp.int32),
            pltpu.VMEM((_CHUNK, H), f32),
            pltpu.VMEM((_CHUNK, H), f32),
            pltpu.VMEM((_CHUNK,), f32),
            pltpu.VMEM((_CHUNK,), f32),
            pltpu.VMEM((N,), f32),
            pltpu.SemaphoreType.DMA,
        ],
        compiler_params=pltpu.CompilerParams(needs_layout_passes=False),
    )
    def sc_gather(mem_hbm, lu_hbm, sidx_hbm, didx_hbm,
                  srows_hbm, drows_hbm, slu_hbm, dlu_hbm,
                  sidx_v, didx_v, srow_v, drow_v, slu_v, dlu_v, lu_v, sem):
        wid = lax.axis_index("s") * _NUM_CORES + lax.axis_index("c")
        pltpu.sync_copy(lu_hbm, lu_v)

        def body(i, _):
            chunk = wid + _NUM_WORKERS * i

            @pl.when(chunk < n_chunks)
            def _():
                base = chunk * _CHUNK
                pltpu.sync_copy(sidx_hbm.at[pl.ds(base, _CHUNK)], sidx_v)
                pltpu.sync_copy(didx_hbm.at[pl.ds(base, _CHUNK)], didx_v)
                cps = [
                    pltpu.async_copy(mem_hbm.at[sidx_v], srow_v, sem),
                    pltpu.async_copy(mem_hbm.at[didx_v], drow_v, sem),
                ]
                for j in range(_CHUNK // 16):
                    sl = pl.ds(j * 16, 16)
                    slu_v[sl] = plsc.load_gather(lu_v, [sidx_v[sl]])
                    dlu_v[sl] = plsc.load_gather(lu_v, [didx_v[sl]])
                for cp in cps:
                    cp.wait()
                pltpu.sync_copy(srow_v, srows_hbm.at[pl.ds(base, _CHUNK)])
                pltpu.sync_copy(drow_v, drows_hbm.at[pl.ds(base, _CHUNK)])
                pltpu.sync_copy(slu_v, slu_hbm.at[pl.ds(base, _CHUNK)])
                pltpu.sync_copy(dlu_v, dlu_hbm.at[pl.ds(base, _CHUNK)])

            return None

        lax.fori_loop(0, iters, body, None)

    return sc_gather


def _assemble_body(type_ref, smask_ref, dmask_ref, ts_ref, slu_ref, dlu_ref,
                   srows_ref, drows_ref, evt_ref, w_ref, b_ref,
                   out_src_ref, out_dst_ref):
    H = srows_ref.shape[1]
    B = srows_ref.shape[0]
    type_col = type_ref[...].astype(jnp.float32)
    sm = smask_ref[...]
    dm = dmask_ref[...]
    t = ts_ref[...]
    w = w_ref[...]
    b = b_ref[...]
    src_embs = srows_ref[...] * sm
    dst_embs = drows_ref[...] * dm
    evt = evt_ref[...]
    src_te = jnp.cos((t - slu_ref[...] * dm) * w + b)
    dst_te = jnp.cos((t - dlu_ref[...] * dm) * w + b)
    type_b = jnp.broadcast_to(type_col, (B, H))

    out_src_ref[:, 0:H] = type_b
    out_src_ref[:, H:2 * H] = src_embs
    out_src_ref[:, 2 * H:3 * H] = dst_embs
    out_src_ref[:, 3 * H:4 * H] = src_te
    out_src_ref[:, 4 * H:5 * H] = evt

    out_dst_ref[:, 0:H] = type_b
    out_dst_ref[:, H:2 * H] = dst_embs
    out_dst_ref[:, 2 * H:3 * H] = src_embs
    out_dst_ref[:, 3 * H:4 * H] = dst_te
    out_dst_ref[:, 4 * H:5 * H] = evt


@functools.lru_cache(maxsize=None)
def _make_assemble(E, H, B=512, interpret=False):
    grid = (E // B,)
    col = pl.BlockSpec((B, 1), lambda i: (i, 0))
    row = pl.BlockSpec((B, H), lambda i: (i, 0))
    const = pl.BlockSpec((1, H), lambda i: (0, 0))
    return pl.pallas_call(
        _assemble_body,
        grid=grid,
        in_specs=[col, col, col, col, col, col, row, row, row, const, const],
        out_specs=[
            pl.BlockSpec((B, 5 * H), lambda i: (i, 0)),
            pl.BlockSpec((B, 5 * H), lambda i: (i, 0)),
        ],
        out_shape=[
            jax.ShapeDtypeStruct((E, 5 * H), jnp.float32),
            jax.ShapeDtypeStruct((E, 5 * H), jnp.float32),
        ],
        compiler_params=pltpu.CompilerParams(
            dimension_semantics=("arbitrary",),
        ),
        interpret=interpret,
    )


def kernel(event_type_ids, src_ids, src_mask, dst_ids, dst_mask,
           event_embeddings, event_timestamps, memory, last_update,
           time_w, time_b):
    E, H = event_embeddings.shape
    N = memory.shape[0]

    sc_gather = _make_sc_gather(E, N, H)
    srows, drows, slu, dlu = sc_gather(
        memory,
        last_update,
        src_ids.astype(jnp.int32),
        dst_ids.astype(jnp.int32),
    )
    slu = slu.reshape(E, 1)
    dlu = dlu.reshape(E, 1)

    assemble = _make_assemble(E, H)
    out_src, out_dst = assemble(
        event_type_ids.astype(jnp.int32).reshape(E, 1),
        src_mask.reshape(E, 1),
        dst_mask.reshape(E, 1),
        event_timestamps.reshape(E, 1),
        slu,
        dlu,
        srows,
        drows,
        event_embeddings,
        time_w.reshape(1, H),
        time_b.reshape(1, H),
    )
    return (out_src, out_dst)


# speedup vs baseline: 4.5115x; 1.1891x over previous
import functools

import jax
import jax.numpy as jnp
from jax import lax
from jax.experimental import pallas as pl
from jax.experimental.pallas import tpu as pltpu
from jax.experimental.pallas import tpu_sc as plsc

_NUM_CORES = 2
_NUM_SUBCORES = 16
_NUM_WORKERS = _NUM_CORES * _NUM_SUBCORES
_CHUNK = 128


@functools.lru_cache(maxsize=None)
def _make_sc_gather(E, N, H):
    n_chunks = E // _CHUNK
    iters = -(-n_chunks // _NUM_WORKERS)
    mesh = plsc.VectorSubcoreMesh(core_axis_name="c", subcore_axis_name="s")
    f32 = jnp.float32

    @functools.partial(
        pl.kernel,
        mesh=mesh,
        out_type=[
            jax.ShapeDtypeStruct((E, H), f32),
            jax.ShapeDtypeStruct((E, H), f32),
            jax.ShapeDtypeStruct((E,), f32),
            jax.ShapeDtypeStruct((E,), f32),
        ],
        scratch_types=[
            pltpu.VMEM((_CHUNK,), jnp.int32),
            pltpu.VMEM((_CHUNK,), jnp.int32),
            pltpu.VMEM((_CHUNK, H), f32),
            pltpu.VMEM((_CHUNK, H), f32),
            pltpu.VMEM((_CHUNK,), f32),
            pltpu.VMEM((_CHUNK,), f32),
            pltpu.VMEM((N,), f32),
            pltpu.SemaphoreType.DMA,
        ],
        compiler_params=pltpu.CompilerParams(needs_layout_passes=False),
    )
    def sc_gather(mem_hbm, lu_hbm, sidx_hbm, didx_hbm,
                  srows_hbm, drows_hbm, slu_hbm, dlu_hbm,
                  sidx_v, didx_v, srow_v, drow_v, slu_v, dlu_v, lu_v, sem):
        wid = lax.axis_index("s") * _NUM_CORES + lax.axis_index("c")
        pltpu.sync_copy(lu_hbm, lu_v)

        def body(i, _):
            chunk = wid + _NUM_WORKERS * i

            @pl.when(chunk < n_chunks)
            def _():
                base = chunk * _CHUNK
                pltpu.sync_copy(sidx_hbm.at[pl.ds(base, _CHUNK)], sidx_v)
                pltpu.sync_copy(didx_hbm.at[pl.ds(base, _CHUNK)], didx_v)
                cps = [
                    pltpu.async_copy(mem_hbm.at[sidx_v], srow_v, sem),
                    pltpu.async_copy(mem_hbm.at[didx_v], drow_v, sem),
                ]
                for j in range(_CHUNK // 16):
                    sl = pl.ds(j * 16, 16)
                    slu_v[sl] = plsc.load_gather(lu_v, [sidx_v[sl]])
                    dlu_v[sl] = plsc.load_gather(lu_v, [didx_v[sl]])
                for cp in cps:
                    cp.wait()
                pltpu.sync_copy(srow_v, srows_hbm.at[pl.ds(base, _CHUNK)])
                pltpu.sync_copy(drow_v, drows_hbm.at[pl.ds(base, _CHUNK)])
                pltpu.sync_copy(slu_v, slu_hbm.at[pl.ds(base, _CHUNK)])
                pltpu.sync_copy(dlu_v, dlu_hbm.at[pl.ds(base, _CHUNK)])

            return None

        lax.fori_loop(0, iters, body, None)

    return sc_gather


def _fast_cos(x):
    inv2pi = jnp.float32(0.15915493667125702)
    magic = jnp.float32(12582912.0)
    c1 = jnp.float32(6.28125)
    c2 = jnp.float32(0.0019353071795864769)
    k = (x * inv2pi + magic) - magic
    r = (x - k * c1) - k * c2
    u = r * r
    p = jnp.float32(1.736913401585966e-09)
    for c in (-2.711337329987122e-07, 2.47734242079983e-05,
              -0.0013887970411328634, 0.041666524363789405,
              -0.4999999177196379, 0.9999999922771011):
        p = p * u + jnp.float32(c)
    return p


def _assemble_body(type_ref, smask_ref, dmask_ref, ts_ref, slu_ref, dlu_ref,
                   srows_ref, drows_ref, evt_ref, w_ref, b_ref,
                   out_src_ref, out_dst_ref):
    H = srows_ref.shape[1]
    B = srows_ref.shape[0]
    type_col = type_ref[...].astype(jnp.float32)
    sm = smask_ref[...]
    dm = dmask_ref[...]
    t = ts_ref[...]
    w = w_ref[...]
    b = b_ref[...]
    src_embs = srows_ref[...] * sm
    dst_embs = drows_ref[...] * dm
    evt = evt_ref[...]
    src_te = _fast_cos((t - slu_ref[...] * dm) * w + b)
    dst_te = _fast_cos((t - dlu_ref[...] * dm) * w + b)
    type_b = jnp.broadcast_to(type_col, (B, H))

    out_src_ref[:, 0:H] = type_b
    out_src_ref[:, H:2 * H] = src_embs
    out_src_ref[:, 2 * H:3 * H] = dst_embs
    out_src_ref[:, 3 * H:4 * H] = src_te
    out_src_ref[:, 4 * H:5 * H] = evt

    out_dst_ref[:, 0:H] = type_b
    out_dst_ref[:, H:2 * H] = dst_embs
    out_dst_ref[:, 2 * H:3 * H] = src_embs
    out_dst_ref[:, 3 * H:4 * H] = dst_te
    out_dst_ref[:, 4 * H:5 * H] = evt


@functools.lru_cache(maxsize=None)
def _make_assemble(E, H, B=512, interpret=False):
    grid = (E // B,)
    col = pl.BlockSpec((B, 1), lambda i: (i, 0))
    row = pl.BlockSpec((B, H), lambda i: (i, 0))
    const = pl.BlockSpec((1, H), lambda i: (0, 0))
    return pl.pallas_call(
        _assemble_body,
        grid=grid,
        in_specs=[col, col, col, col, col, col, row, row, row, const, const],
        out_specs=[
            pl.BlockSpec((B, 5 * H), lambda i: (i, 0)),
            pl.BlockSpec((B, 5 * H), lambda i: (i, 0)),
        ],
        out_shape=[
            jax.ShapeDtypeStruct((E, 5 * H), jnp.float32),
            jax.ShapeDtypeStruct((E, 5 * H), jnp.float32),
        ],
        compiler_params=pltpu.CompilerParams(
            dimension_semantics=("arbitrary",),
        ),
        interpret=interpret,
    )


def kernel(event_type_ids, src_ids, src_mask, dst_ids, dst_mask,
           event_embeddings, event_timestamps, memory, last_update,
           time_w, time_b):
    E, H = event_embeddings.shape
    N = memory.shape[0]

    sc_gather = _make_sc_gather(E, N, H)
    srows, drows, slu, dlu = sc_gather(
        memory,
        last_update,
        src_ids.astype(jnp.int32),
        dst_ids.astype(jnp.int32),
    )
    slu = slu.reshape(E, 1)
    dlu = dlu.reshape(E, 1)

    assemble = _make_assemble(E, H)
    out_src, out_dst = assemble(
        event_type_ids.astype(jnp.int32).reshape(E, 1),
        src_mask.reshape(E, 1),
        dst_mask.reshape(E, 1),
        event_timestamps.reshape(E, 1),
        slu,
        dlu,
        srows,
        drows,
        event_embeddings,
        time_w.reshape(1, H),
        time_b.reshape(1, H),
    )
    return (out_src, out_dst)
